# baseline (speedup 1.0000x reference)
import functools

import jax
import jax.numpy as jnp
from jax import lax
from jax.experimental import pallas as pl
from jax.experimental.pallas import tpu as pltpu
from jax.experimental.pallas import tpu_sc as plsc

EMB = 64
BATCH = 4096
L = 16
NC = 2
NS = 16
NW = NC * NS
BPW = BATCH // NW


def _sc_gather_dot(user_emb, food_emb, user_bias, food_bias, uid, fid):
    mesh = plsc.VectorSubcoreMesh(core_axis_name="c", subcore_axis_name="s")

    @functools.partial(
        pl.kernel,
        mesh=mesh,
        compiler_params=pltpu.CompilerParams(use_tc_tiling_on_sc=False),
        out_type=(
            jax.ShapeDtypeStruct((NW, L), jnp.float32),
            jax.ShapeDtypeStruct((BATCH,), jnp.float32),
            jax.ShapeDtypeStruct((BATCH,), jnp.float32),
        ),
        scratch_types=[
            pltpu.VMEM((BPW,), jnp.int32),
            pltpu.VMEM((BPW,), jnp.int32),
            pltpu.VMEM((BPW, EMB), jnp.float32),
            pltpu.VMEM((BPW, EMB), jnp.float32),
            pltpu.VMEM((BPW,), jnp.float32),
            pltpu.VMEM((BPW,), jnp.float32),
            pltpu.VMEM((L,), jnp.float32),
            pltpu.SemaphoreType.DMA,
            pltpu.SemaphoreType.DMA,
            pltpu.SemaphoreType.DMA,
            pltpu.SemaphoreType.DMA,
        ],
    )
    def k(uemb, femb, ubias, fbias, uid_h, fid_h,
          part_out, ub_out, fb_out,
          uidx_v, fidx_v, urows_v, frows_v, ub_v, fb_v, acc_v,
          sem_u, sem_f, sem_ub, sem_fb):
        wid = lax.axis_index("s") * NC + lax.axis_index("c")
        base = wid * BPW
        pltpu.sync_copy(uid_h.at[pl.ds(base, BPW)], uidx_v)
        pltpu.sync_copy(fid_h.at[pl.ds(base, BPW)], fidx_v)
        cu = pltpu.async_copy(uemb.at[uidx_v], urows_v, sem_u)
        cf = pltpu.async_copy(femb.at[fidx_v], frows_v, sem_f)
        cub = pltpu.async_copy(ubias.at[uidx_v], ub_v, sem_ub)
        cfb = pltpu.async_copy(fbias.at[fidx_v], fb_v, sem_fb)
        cu.wait()
        cf.wait()

        def body(i, accs):
            a0, a1, a2, a3 = accs
            a0 = a0 + urows_v[i, pl.ds(0 * L, L)] * frows_v[i, pl.ds(0 * L, L)]
            a1 = a1 + urows_v[i, pl.ds(1 * L, L)] * frows_v[i, pl.ds(1 * L, L)]
            a2 = a2 + urows_v[i, pl.ds(2 * L, L)] * frows_v[i, pl.ds(2 * L, L)]
            a3 = a3 + urows_v[i, pl.ds(3 * L, L)] * frows_v[i, pl.ds(3 * L, L)]
            return (a0, a1, a2, a3)

        z = jnp.zeros((L,), jnp.float32)
        a0, a1, a2, a3 = lax.fori_loop(0, BPW, body, (z, z, z, z))
        acc_v[...] = (a0 + a1) + (a2 + a3)
        pltpu.sync_copy(acc_v, part_out.at[wid])
        cub.wait()
        cfb.wait()
        pltpu.sync_copy(ub_v, ub_out.at[pl.ds(base, BPW)])
        pltpu.sync_copy(fb_v, fb_out.at[pl.ds(base, BPW)])

    return k(user_emb, food_emb, user_bias, food_bias, uid, fid)


def _tc_mlp(partials, ub, fb, w1r, b1r, w2, b2r, w3r, b3r):
    def body(p_ref, ub_ref, fb_ref, w1_ref, b1_ref, w2_ref, b2_ref,
             w3_ref, b3_ref, out_ref):
        s = jnp.sum(p_ref[...])
        x = s + ub_ref[...] + fb_ref[...]
        h1 = jnp.maximum(x * w1_ref[...] + b1_ref[...], 0.0)
        h2 = jnp.maximum(
            jnp.dot(h1, w2_ref[...], preferred_element_type=jnp.float32)
            + b2_ref[...], 0.0)
        zz = jnp.sum(h2 * w3_ref[...], axis=1, keepdims=True) + b3_ref[...]
        out_ref[...] = 1.0 / (1.0 + jnp.exp(-zz))

    return pl.pallas_call(
        body,
        out_shape=jax.ShapeDtypeStruct((BATCH, 1), jnp.float32),
    )(partials, ub, fb, w1r, b1r, w2, b2r, w3r, b3r)


def kernel(inputs, user_emb, user_bias, food_emb, food_bias, W1, b1, W2, b2, W3, b3):
    idx = inputs.astype(jnp.int32)
    uid = idx[:, 0]
    fid = idx[:, 1]
    partials, ub, fb = _sc_gather_dot(
        user_emb, food_emb, user_bias.reshape(-1), food_bias.reshape(-1),
        uid, fid)
    return _tc_mlp(
        partials, ub.reshape(BATCH, 1), fb.reshape(BATCH, 1),
        W1.reshape(1, 128), b1.reshape(1, 128),
        W2, b2.reshape(1, 64),
        W3.reshape(1, 64), b3.reshape(1, 1))
